# Initial kernel scaffold; baseline (speedup 1.0000x reference)
import jax
import jax.numpy as jnp
from jax.experimental import pallas as pl


def kernel(hidden_states, weight):
    raise NotImplementedError("write your pallas kernel here")


# speedup vs baseline: 3.9544x; 3.9544x over previous
import jax
import jax.numpy as jnp
from jax.experimental import pallas as pl
from jax.experimental.pallas import tpu as pltpu

_NE = 256
_NG = 8
_GS = 32
_TKG = 4
_TK = 8
_H = 2048
_T = 8192
_ALPHA = 0.001
_BT = 256
_NBLK = _T // _BT
_NEG = float("-inf")


def _gate_kernel(x_ref, w_ref, idx_ref, wt_ref, aux_ref, hist_ref, psum_ref):
    i = pl.program_id(0)

    @pl.when(i == 0)
    def _init():
        hist_ref[...] = jnp.zeros_like(hist_ref)
        psum_ref[...] = jnp.zeros_like(psum_ref)

    x = x_ref[...]
    w = w_ref[...]
    logits = jax.lax.dot_general(
        w, x, (((1,), (1,)), ((), ())),
        preferred_element_type=jnp.float32)

    m = jnp.max(logits, axis=0, keepdims=True)
    p = jnp.exp(logits - m)
    s = jnp.sum(p, axis=0, keepdims=True)
    psum_ref[...] += jnp.sum(p / s, axis=1, keepdims=True)

    l3 = logits.reshape(_NG, _GS, _BT)
    sub_iota = jax.lax.broadcasted_iota(jnp.int32, (_NG, _GS, _BT), 1)
    grp_base = jax.lax.broadcasted_iota(jnp.int32, (_NG, _BT), 0) * _GS
    cand_mask = jnp.zeros((_NG, _GS, _BT), jnp.bool_)
    work = l3
    cvals = []
    cidxs = []
    for _ in range(_TKG):
        gm = jnp.max(work, axis=1, keepdims=True)
        ism = work == gm
        wi = jnp.min(jnp.where(ism, sub_iota, _GS), axis=1, keepdims=True)
        winner = sub_iota == wi
        cand_mask = jnp.logical_or(cand_mask, winner)
        work = jnp.where(winner, _NEG, work)
        cvals.append(gm.reshape(_NG, _BT))
        cidxs.append(wi.reshape(_NG, _BT) + grp_base)
    cval = jnp.concatenate(cvals, axis=0)
    cidx = jnp.concatenate(cidxs, axis=0)

    vals = []
    idxs = []
    for _ in range(_TK):
        mv = jnp.max(cval, axis=0, keepdims=True)
        eid = jnp.min(jnp.where(cval == mv, cidx, _NE), axis=0, keepdims=True)
        sel = jnp.logical_and(cval == mv, cidx == eid)
        cval = jnp.where(sel, _NEG, cval)
        vals.append(mv)
        idxs.append(eid)
    v8 = jnp.concatenate(vals, axis=0)
    i8 = jnp.concatenate(idxs, axis=0)
    wsum = jnp.sum(v8, axis=0, keepdims=True) + 1e-20
    wt_ref[...] = v8 / wsum
    idx_ref[...] = i8

    thresh = vals[-1].reshape(1, 1, _BT)
    sel_full = jnp.logical_and(cand_mask, l3 >= thresh)
    hist_ref[...] += jnp.sum(
        sel_full.astype(jnp.float32).reshape(_NE, _BT), axis=1, keepdims=True)

    @pl.when(i == _NBLK - 1)
    def _fin():
        aux_ref[...] = jnp.sum(
            hist_ref[...] * psum_ref[...], axis=(0, 1), keepdims=True) * (
            _ALPHA / (_TK * _T * _T))


@jax.jit
def kernel(hidden_states, weight):
    idx_t, wt_t, aux = pl.pallas_call(
        _gate_kernel,
        grid=(_NBLK,),
        in_specs=[
            pl.BlockSpec((_BT, _H), lambda i: (i, 0)),
            pl.BlockSpec((_NE, _H), lambda i: (0, 0)),
        ],
        out_specs=[
            pl.BlockSpec((_TK, _BT), lambda i: (0, i)),
            pl.BlockSpec((_TK, _BT), lambda i: (0, i)),
            pl.BlockSpec((1, 1), lambda i: (0, 0)),
        ],
        out_shape=[
            jax.ShapeDtypeStruct((_TK, _T), jnp.int32),
            jax.ShapeDtypeStruct((_TK, _T), jnp.float32),
            jax.ShapeDtypeStruct((1, 1), jnp.float32),
        ],
        scratch_shapes=[
            pltpu.VMEM((_NE, 1), jnp.float32),
            pltpu.VMEM((_NE, 1), jnp.float32),
        ],
        compiler_params=pltpu.CompilerParams(
            dimension_semantics=("arbitrary",)),
    )(hidden_states, weight)
    return idx_t.T, wt_t.T, aux[0, 0]


# speedup vs baseline: 4.1540x; 1.0505x over previous
import jax
import jax.numpy as jnp
from jax.experimental import pallas as pl
from jax.experimental.pallas import tpu as pltpu

_NE = 256
_NG = 8
_GS = 32
_TKG = 4
_TK = 8
_H = 2048
_T = 8192
_ALPHA = 0.001
_BT = 256
_NBLK = _T // _BT
_NEG = float("-inf")


def _gate_kernel(x_ref, w_ref, idx_ref, wt_ref, aux_ref, hist_ref, psum_ref):
    i = pl.program_id(0)

    @pl.when(i == 0)
    def _init():
        hist_ref[...] = jnp.zeros_like(hist_ref)
        psum_ref[...] = jnp.zeros_like(psum_ref)

    x = x_ref[...]
    w = w_ref[...]
    logits = jax.lax.dot_general(
        w, x, (((1,), (1,)), ((), ())),
        preferred_element_type=jnp.float32)

    l3 = logits.reshape(_NG, _GS, _BT)
    sub_iota = jax.lax.broadcasted_iota(jnp.int32, (_NG, _GS, _BT), 1)
    grp_base = jax.lax.broadcasted_iota(jnp.int32, (_NG, _BT), 0) * _GS
    work = l3
    cvals = []
    cidxs = []
    for _ in range(_TKG):
        gm = jnp.max(work, axis=1, keepdims=True)
        ism = work == gm
        wi = jnp.min(jnp.where(ism, sub_iota, _GS), axis=1, keepdims=True)
        work = jnp.where(sub_iota == wi, _NEG, work)
        cvals.append(gm.reshape(_NG, _BT))
        cidxs.append(wi.reshape(_NG, _BT) + grp_base)
    cval = jnp.concatenate(cvals, axis=0)
    cidx = jnp.concatenate(cidxs, axis=0)

    vals = []
    idxs = []
    for _ in range(_TK):
        mv = jnp.max(cval, axis=0, keepdims=True)
        ism = cval == mv
        eid = jnp.min(jnp.where(ism, cidx, _NE), axis=0, keepdims=True)
        cval = jnp.where(jnp.logical_and(ism, cidx == eid), _NEG, cval)
        vals.append(mv)
        idxs.append(eid)
    v8 = jnp.concatenate(vals, axis=0)
    i8 = jnp.concatenate(idxs, axis=0)
    wsum = jnp.sum(v8, axis=0, keepdims=True) + 1e-20
    wt_ref[...] = v8 / wsum
    idx_ref[...] = i8

    p = jnp.exp(logits - vals[0])
    s = jnp.sum(p, axis=0, keepdims=True)
    psum_ref[...] += p * (1.0 / s)

    selm = jnp.logical_and(work == _NEG, l3 >= vals[-1].reshape(1, 1, _BT))
    hist_ref[...] += selm.astype(jnp.float32).reshape(_NE, _BT)

    @pl.when(i == _NBLK - 1)
    def _fin():
        hv = jnp.sum(hist_ref[...], axis=1, keepdims=True)
        pv = jnp.sum(psum_ref[...], axis=1, keepdims=True)
        aux_ref[...] = jnp.sum(hv * pv, axis=(0, 1), keepdims=True) * (
            _ALPHA / (_TK * _T * _T))


@jax.jit
def kernel(hidden_states, weight):
    idx_t, wt_t, aux = pl.pallas_call(
        _gate_kernel,
        grid=(_NBLK,),
        in_specs=[
            pl.BlockSpec((_BT, _H), lambda i: (i, 0)),
            pl.BlockSpec((_NE, _H), lambda i: (0, 0)),
        ],
        out_specs=[
            pl.BlockSpec((_TK, _BT), lambda i: (0, i)),
            pl.BlockSpec((_TK, _BT), lambda i: (0, i)),
            pl.BlockSpec((1, 1), lambda i: (0, 0)),
        ],
        out_shape=[
            jax.ShapeDtypeStruct((_TK, _T), jnp.int32),
            jax.ShapeDtypeStruct((_TK, _T), jnp.float32),
            jax.ShapeDtypeStruct((1, 1), jnp.float32),
        ],
        scratch_shapes=[
            pltpu.VMEM((_NE, _BT), jnp.float32),
            pltpu.VMEM((_NE, _BT), jnp.float32),
        ],
        compiler_params=pltpu.CompilerParams(
            dimension_semantics=("arbitrary",)),
    )(hidden_states, weight)
    return idx_t.T, wt_t.T, aux[0, 0]


# speedup vs baseline: 4.3415x; 1.0452x over previous
import jax
import jax.numpy as jnp
from jax.experimental import pallas as pl
from jax.experimental.pallas import tpu as pltpu

_NE = 256
_NG = 8
_GS = 32
_TKG = 4
_TK = 8
_H = 2048
_T = 8192
_ALPHA = 0.001
_BT = 256
_NBLK = _T // _BT
_NEG = float("-inf")


def _gate_kernel(x_ref, w_ref, idx_ref, wt_ref, aux_ref,
                 lbuf_ref, hist_ref, psum_ref):
    i = pl.program_id(0)
    last = pl.num_programs(0) - 1

    logits = lbuf_ref[...]

    x = x_ref[...]
    w = w_ref[...]
    lbuf_ref[...] = jax.lax.dot_general(
        w, x, (((1,), (1,)), ((), ())),
        preferred_element_type=jnp.float32)

    l3 = logits.reshape(_NG, _GS, _BT)
    sub_iota = jax.lax.broadcasted_iota(jnp.int32, (_NG, _GS, _BT), 1)
    grp_base = jax.lax.broadcasted_iota(jnp.int32, (_NG, _BT), 0) * _GS
    work = l3
    cvals = []
    cidxs = []
    for _ in range(_TKG):
        gm = jnp.max(work, axis=1, keepdims=True)
        ism = work == gm
        wi = jnp.min(jnp.where(ism, sub_iota, _GS), axis=1, keepdims=True)
        work = jnp.where(sub_iota == wi, _NEG, work)
        cvals.append(gm.reshape(_NG, _BT))
        cidxs.append(wi.reshape(_NG, _BT) + grp_base)
    cval = jnp.concatenate(cvals, axis=0)
    cidx = jnp.concatenate(cidxs, axis=0)

    vals = []
    idxs = []
    for _ in range(_TK):
        mv = jnp.max(cval, axis=0, keepdims=True)
        ism = cval == mv
        eid = jnp.min(jnp.where(ism, cidx, _NE), axis=0, keepdims=True)
        cval = jnp.where(jnp.logical_and(ism, cidx == eid), _NEG, cval)
        vals.append(mv)
        idxs.append(eid)
    v8 = jnp.concatenate(vals, axis=0)
    i8 = jnp.concatenate(idxs, axis=0)
    wsum = jnp.sum(v8, axis=0, keepdims=True) + 1e-20
    wt_ref[...] = v8 / wsum
    idx_ref[...] = i8

    p = jnp.exp(logits - vals[0])
    s = jnp.sum(p, axis=0, keepdims=True)
    pnorm = p * (1.0 / s)

    selm = jnp.logical_and(work == _NEG, l3 >= vals[-1].reshape(1, 1, _BT))
    hcontrib = selm.astype(jnp.float32).reshape(_NE, _BT)

    @pl.when(i == 0)
    def _init():
        hist_ref[...] = jnp.zeros_like(hist_ref)
        psum_ref[...] = jnp.zeros_like(psum_ref)

    @pl.when(i > 0)
    def _acc():
        hist_ref[...] += hcontrib
        psum_ref[...] += pnorm

    @pl.when(i == last)
    def _fin():
        hv = jnp.sum(hist_ref[...], axis=1, keepdims=True)
        pv = jnp.sum(psum_ref[...], axis=1, keepdims=True)
        aux_ref[...] = jnp.sum(hv * pv, axis=(0, 1), keepdims=True) * (
            _ALPHA / (_TK * _T * _T))


@jax.jit
def kernel(hidden_states, weight):
    idx_t, wt_t, aux = pl.pallas_call(
        _gate_kernel,
        grid=(_NBLK + 1,),
        in_specs=[
            pl.BlockSpec((_BT, _H), lambda i: (jnp.minimum(i, _NBLK - 1), 0)),
            pl.BlockSpec((_NE, _H), lambda i: (0, 0)),
        ],
        out_specs=[
            pl.BlockSpec((_TK, _BT), lambda i: (0, jnp.maximum(i - 1, 0))),
            pl.BlockSpec((_TK, _BT), lambda i: (0, jnp.maximum(i - 1, 0))),
            pl.BlockSpec((1, 1), lambda i: (0, 0)),
        ],
        out_shape=[
            jax.ShapeDtypeStruct((_TK, _T), jnp.int32),
            jax.ShapeDtypeStruct((_TK, _T), jnp.float32),
            jax.ShapeDtypeStruct((1, 1), jnp.float32),
        ],
        scratch_shapes=[
            pltpu.VMEM((_NE, _BT), jnp.float32),
            pltpu.VMEM((_NE, _BT), jnp.float32),
            pltpu.VMEM((_NE, _BT), jnp.float32),
        ],
        compiler_params=pltpu.CompilerParams(
            dimension_semantics=("arbitrary",)),
    )(hidden_states, weight)
    return idx_t.T, wt_t.T, aux[0, 0]


# speedup vs baseline: 5.1757x; 1.1921x over previous
import jax
import jax.numpy as jnp
from jax.experimental import pallas as pl
from jax.experimental.pallas import tpu as pltpu

_NE = 256
_NG = 8
_GS = 32
_TKG = 4
_TK = 8
_H = 2048
_T = 8192
_ALPHA = 0.001
_BT = 512
_NBLK = _T // _BT
_NEG = float("-inf")


def _gate_kernel(x_ref, w_ref, idx_ref, wt_ref, aux_ref,
                 lbuf_ref, hist_ref, psum_ref):
    i = pl.program_id(0)
    last = pl.num_programs(0) - 1

    logits = lbuf_ref[...]

    x = x_ref[...]
    w = w_ref[...]
    lbuf_ref[...] = jax.lax.dot_general(
        w, x, (((1,), (1,)), ((), ())),
        preferred_element_type=jnp.float32)

    l3 = logits.reshape(_NG, _GS, _BT)
    sub_iota = jax.lax.broadcasted_iota(jnp.int32, (_NG, _GS, _BT), 1)
    grp_base = jax.lax.broadcasted_iota(jnp.int32, (_NG, _BT), 0) * _GS
    work = l3
    cvals = []
    cidxs = []
    for _ in range(_TKG):
        gm = jnp.max(work, axis=1, keepdims=True)
        ism = work == gm
        wi = jnp.min(jnp.where(ism, sub_iota, _GS), axis=1, keepdims=True)
        work = jnp.where(sub_iota == wi, _NEG, work)
        cvals.append(gm.reshape(_NG, _BT))
        cidxs.append(wi.reshape(_NG, _BT) + grp_base)
    cval = jnp.concatenate(cvals, axis=0)
    cidx = jnp.concatenate(cidxs, axis=0)

    vals = []
    idxs = []
    for _ in range(_TK):
        mv = jnp.max(cval, axis=0, keepdims=True)
        ism = cval == mv
        eid = jnp.min(jnp.where(ism, cidx, _NE), axis=0, keepdims=True)
        cval = jnp.where(jnp.logical_and(ism, cidx == eid), _NEG, cval)
        vals.append(mv)
        idxs.append(eid)
    v8 = jnp.concatenate(vals, axis=0)
    i8 = jnp.concatenate(idxs, axis=0)
    wsum = jnp.sum(v8, axis=0, keepdims=True) + 1e-20
    wt_ref[...] = v8 / wsum
    idx_ref[...] = i8

    p = jnp.exp(logits - vals[0])
    s = jnp.sum(p, axis=0, keepdims=True)
    pnorm = p * (1.0 / s)

    selm = jnp.logical_and(work == _NEG, l3 >= vals[-1].reshape(1, 1, _BT))
    hcontrib = selm.astype(jnp.float32).reshape(_NE, _BT)

    @pl.when(i == 0)
    def _init():
        hist_ref[...] = jnp.zeros_like(hist_ref)
        psum_ref[...] = jnp.zeros_like(psum_ref)

    @pl.when(i > 0)
    def _acc():
        hist_ref[...] += hcontrib
        psum_ref[...] += pnorm

    @pl.when(i == last)
    def _fin():
        hv = jnp.sum(hist_ref[...], axis=1, keepdims=True)
        pv = jnp.sum(psum_ref[...], axis=1, keepdims=True)
        aux_ref[...] = jnp.sum(hv * pv, axis=(0, 1), keepdims=True) * (
            _ALPHA / (_TK * _T * _T))


@jax.jit
def kernel(hidden_states, weight):
    idx_t, wt_t, aux = pl.pallas_call(
        _gate_kernel,
        grid=(_NBLK + 1,),
        in_specs=[
            pl.BlockSpec((_BT, _H), lambda i: (jnp.minimum(i, _NBLK - 1), 0)),
            pl.BlockSpec((_NE, _H), lambda i: (0, 0)),
        ],
        out_specs=[
            pl.BlockSpec((_TK, _BT), lambda i: (0, jnp.maximum(i - 1, 0))),
            pl.BlockSpec((_TK, _BT), lambda i: (0, jnp.maximum(i - 1, 0))),
            pl.BlockSpec((1, 1), lambda i: (0, 0)),
        ],
        out_shape=[
            jax.ShapeDtypeStruct((_TK, _T), jnp.int32),
            jax.ShapeDtypeStruct((_TK, _T), jnp.float32),
            jax.ShapeDtypeStruct((1, 1), jnp.float32),
        ],
        scratch_shapes=[
            pltpu.VMEM((_NE, _BT), jnp.float32),
            pltpu.VMEM((_NE, _BT), jnp.float32),
            pltpu.VMEM((_NE, _BT), jnp.float32),
        ],
        compiler_params=pltpu.CompilerParams(
            dimension_semantics=("arbitrary",)),
    )(hidden_states, weight)
    return idx_t.T, wt_t.T, aux[0, 0]


# speedup vs baseline: 5.1946x; 1.0037x over previous
import jax
import jax.numpy as jnp
from jax.experimental import pallas as pl
from jax.experimental.pallas import tpu as pltpu

_NE = 256
_NG = 8
_GS = 32
_TKG = 4
_TK = 8
_H = 2048
_T = 8192
_ALPHA = 0.001
_BT = 1024
_NBLK = _T // _BT
_NEG = float("-inf")


def _gate_kernel(x_ref, w_ref, idx_ref, wt_ref, aux_ref,
                 lbuf_ref, hist_ref, psum_ref):
    i = pl.program_id(0)
    last = pl.num_programs(0) - 1

    logits = lbuf_ref[...]

    x = x_ref[...]
    w = w_ref[...]
    lbuf_ref[...] = jax.lax.dot_general(
        w, x, (((1,), (1,)), ((), ())),
        preferred_element_type=jnp.float32)

    l3 = logits.reshape(_NG, _GS, _BT)
    sub_iota = jax.lax.broadcasted_iota(jnp.int32, (_NG, _GS, _BT), 1)
    grp_base = jax.lax.broadcasted_iota(jnp.int32, (_NG, _BT), 0) * _GS
    work = l3
    cvals = []
    cidxs = []
    for _ in range(_TKG):
        gm = jnp.max(work, axis=1, keepdims=True)
        ism = work == gm
        wi = jnp.min(jnp.where(ism, sub_iota, _GS), axis=1, keepdims=True)
        work = jnp.where(sub_iota == wi, _NEG, work)
        cvals.append(gm.reshape(_NG, _BT))
        cidxs.append(wi.reshape(_NG, _BT) + grp_base)
    cval = jnp.concatenate(cvals, axis=0)
    cidx = jnp.concatenate(cidxs, axis=0)

    vals = []
    idxs = []
    for _ in range(_TK):
        mv = jnp.max(cval, axis=0, keepdims=True)
        ism = cval == mv
        eid = jnp.min(jnp.where(ism, cidx, _NE), axis=0, keepdims=True)
        cval = jnp.where(jnp.logical_and(ism, cidx == eid), _NEG, cval)
        vals.append(mv)
        idxs.append(eid)
    v8 = jnp.concatenate(vals, axis=0)
    i8 = jnp.concatenate(idxs, axis=0)
    wsum = jnp.sum(v8, axis=0, keepdims=True) + 1e-20
    wt_ref[...] = v8 / wsum
    idx_ref[...] = i8

    p = jnp.exp(logits - vals[0])
    s = jnp.sum(p, axis=0, keepdims=True)
    pnorm = p * (1.0 / s)

    selm = jnp.logical_and(work == _NEG, l3 >= vals[-1].reshape(1, 1, _BT))
    hcontrib = selm.astype(jnp.float32).reshape(_NE, _BT)

    @pl.when(i == 0)
    def _init():
        hist_ref[...] = jnp.zeros_like(hist_ref)
        psum_ref[...] = jnp.zeros_like(psum_ref)

    @pl.when(i > 0)
    def _acc():
        hist_ref[...] += hcontrib
        psum_ref[...] += pnorm

    @pl.when(i == last)
    def _fin():
        hv = jnp.sum(hist_ref[...], axis=1, keepdims=True)
        pv = jnp.sum(psum_ref[...], axis=1, keepdims=True)
        aux_ref[...] = jnp.sum(hv * pv, axis=(0, 1), keepdims=True) * (
            _ALPHA / (_TK * _T * _T))


@jax.jit
def kernel(hidden_states, weight):
    idx_t, wt_t, aux = pl.pallas_call(
        _gate_kernel,
        grid=(_NBLK + 1,),
        in_specs=[
            pl.BlockSpec((_BT, _H), lambda i: (jnp.minimum(i, _NBLK - 1), 0)),
            pl.BlockSpec((_NE, _H), lambda i: (0, 0)),
        ],
        out_specs=[
            pl.BlockSpec((_TK, _BT), lambda i: (0, jnp.maximum(i - 1, 0))),
            pl.BlockSpec((_TK, _BT), lambda i: (0, jnp.maximum(i - 1, 0))),
            pl.BlockSpec((1, 1), lambda i: (0, 0)),
        ],
        out_shape=[
            jax.ShapeDtypeStruct((_TK, _T), jnp.int32),
            jax.ShapeDtypeStruct((_TK, _T), jnp.float32),
            jax.ShapeDtypeStruct((1, 1), jnp.float32),
        ],
        scratch_shapes=[
            pltpu.VMEM((_NE, _BT), jnp.float32),
            pltpu.VMEM((_NE, _BT), jnp.float32),
            pltpu.VMEM((_NE, _BT), jnp.float32),
        ],
        compiler_params=pltpu.CompilerParams(
            dimension_semantics=("arbitrary",)),
    )(hidden_states, weight)
    return idx_t.T, wt_t.T, aux[0, 0]
